# baseline (speedup 1.0000x reference)
import functools

import jax
import jax.numpy as jnp
from jax import lax
from jax.experimental import pallas as pl
from jax.experimental.pallas import tpu as pltpu
from jax.experimental.pallas import tpu_sc as plsc

VOCAB = 1000000
EMBED = 32
BATCH = 4096
SEQ = 200

NC = 2
NS = 16
NW = NC * NS
B_PW = BATCH // NW

VBLK = VOCAB // 16
BLK_PW = VBLK // NW
BLK_REM = VBLK - BLK_PW * NW
CH = 39
VCH = CH * 16
NCHF = 50


def _mesh():
    return plsc.VectorSubcoreMesh(core_axis_name="c", subcore_axis_name="s")


def _wid():
    return lax.axis_index("s") * NC + lax.axis_index("c")


@functools.partial(
    pl.kernel,
    out_type=jax.ShapeDtypeStruct((VOCAB, EMBED), jnp.float32),
    mesh=_mesh(),
    scratch_types=[
        pltpu.VMEM((EMBED, VCH), jnp.float32),
        pltpu.VMEM((VCH, EMBED), jnp.float32),
        pltpu.VMEM((EMBED, 16), jnp.float32),
    ],
    compiler_params=pltpu.CompilerParams(
        use_tc_tiling_on_sc=False, needs_layout_passes=False
    ),
)
def _transpose_table(tt_hbm, rows_hbm, in_v, out_v, tail_v):
    wid = _wid()
    base = wid * BLK_PW + jnp.minimum(wid, BLK_REM)
    nblk = BLK_PW + jnp.where(wid < BLK_REM, 1, 0)
    iota = lax.iota(jnp.int32, 16)
    io_lo = iota
    io_hi = iota + 16

    def chunk_body(ci, carry):
        voff = (base + ci * CH) * 16
        for e in range(EMBED):
            pltpu.sync_copy(tt_hbm.at[pl.ds(e * VOCAB + voff, VCH)], in_v.at[e])
        for blk in range(CH):
            for vi in range(16):
                col = jnp.full((16,), blk * 16 + vi, jnp.int32)
                lo = plsc.load_gather(in_v, [io_lo, col])
                hi = plsc.load_gather(in_v, [io_hi, col])
                out_v[blk * 16 + vi, 0:16] = lo
                out_v[blk * 16 + vi, 16:32] = hi
        pltpu.sync_copy(out_v, rows_hbm.at[pl.ds(voff, VCH)])
        return carry

    lax.fori_loop(0, NCHF, chunk_body, 0)

    def tail_body(ti, carry):
        voff = (base + NCHF * CH + ti) * 16
        for e in range(EMBED):
            pltpu.sync_copy(tt_hbm.at[pl.ds(e * VOCAB + voff, 16)], tail_v.at[e])
        for vi in range(16):
            col = jnp.full((16,), vi, jnp.int32)
            lo = plsc.load_gather(tail_v, [io_lo, col])
            hi = plsc.load_gather(tail_v, [io_hi, col])
            out_v[vi, 0:16] = lo
            out_v[vi, 16:32] = hi
        pltpu.sync_copy(out_v.at[pl.ds(0, 16)], rows_hbm.at[pl.ds(voff, 16)])
        return carry

    lax.fori_loop(0, nblk - NCHF * CH, tail_body, 0)


@functools.partial(
    pl.kernel,
    out_type=jax.ShapeDtypeStruct((SEQ, 4, NW, 8, 128), jnp.float32),
    mesh=_mesh(),
    scratch_types=[
        pltpu.VMEM((SEQ, B_PW), jnp.int32),
        pltpu.VMEM((2, B_PW, EMBED), jnp.float32),
        pltpu.VMEM((2, 4, 8, 128), jnp.float32),
        pltpu.SemaphoreType.DMA,
        pltpu.SemaphoreType.DMA,
    ],
    compiler_params=pltpu.CompilerParams(
        use_tc_tiling_on_sc=False, needs_layout_passes=False
    ),
)
def _gather(xt_hbm, rows_hbm, out_hbm, idx_v, rows_v, t5_v, gsem, osem):
    wid = _wid()
    iota = lax.iota(jnp.int32, 16)
    pltpu.sync_copy(xt_hbm.at[:, pl.ds(wid * B_PW, B_PW)], idx_v)
    pltpu.async_copy(rows_hbm.at[idx_v.at[0]], rows_v.at[0], gsem)

    def s_body(t, carry):
        for b in range(2):
            s = 2 * t + b
            nxt = s + 1

            @pl.when(nxt < SEQ)
            def _():
                pltpu.async_copy(
                    rows_hbm.at[idx_v.at[nxt]], rows_v.at[1 - b], gsem
                )

            pltpu.make_async_copy(
                rows_hbm.at[idx_v.at[s]], rows_v.at[b], gsem
            ).wait()

            @pl.when(s >= 2)
            def _():
                pltpu.make_async_copy(
                    t5_v.at[b], out_hbm.at[s, :, wid], osem
                ).wait()

            for e in range(EMBED):
                e_splat = jnp.full((16,), e, jnp.int32)
                for h in range(8):
                    g = plsc.load_gather(rows_v.at[b], [iota + h * 16, e_splat])
                    t5_v[b, e // 8, e % 8, h * 16:(h + 1) * 16] = g
            pltpu.async_copy(t5_v.at[b], out_hbm.at[s, :, wid], osem)
        return carry

    lax.fori_loop(0, SEQ // 2, s_body, 0)
    for b in range(2):
        pltpu.make_async_copy(
            t5_v.at[b], out_hbm.at[SEQ - 2 + b, :, wid], osem
        ).wait()


def kernel(x, table):
    rows = _transpose_table(table.T.reshape(-1))
    out5 = _gather(x.T.astype(jnp.int32), rows)
    o = out5.transpose(2, 4, 0, 1, 3)
    return o.reshape(BATCH, SEQ, EMBED)


# speedup vs baseline: 4.6012x; 4.6012x over previous
import functools

import jax
import jax.numpy as jnp
from jax import lax
from jax.experimental import pallas as pl
from jax.experimental.pallas import tpu as pltpu
from jax.experimental.pallas import tpu_sc as plsc

VOCAB = 1000000
EMBED = 32

NC = 2
NS = 16
NW = NC * NS
B_TOTAL = 4096 * 200
ROWS_PW = B_TOTAL // NW
IPS = 128
K = 10
RPB = K * IPS
NBLK = ROWS_PW // RPB
NROWCH = ROWS_PW // IPS


@functools.partial(
    pl.kernel,
    out_type=jax.ShapeDtypeStruct((B_TOTAL, EMBED), jnp.float32),
    mesh=plsc.VectorSubcoreMesh(core_axis_name="c", subcore_axis_name="s"),
    scratch_types=[
        pltpu.VMEM((NROWCH, IPS), jnp.int32),
        pltpu.VMEM((RPB, EMBED), jnp.float32),
        pltpu.SemaphoreType.DMA,
    ],
    compiler_params=pltpu.CompilerParams(use_tc_tiling_on_sc=False),
)
def _emb_lookup(x_hbm, table_hbm, out_hbm, idx_v, rows_v, gsem):
    wid = lax.axis_index("s") * NC + lax.axis_index("c")
    pltpu.sync_copy(x_hbm.at[wid], idx_v)
    out_base = wid * ROWS_PW

    def blk_body(blk, carry):
        descs = [
            pltpu.async_copy(
                table_hbm.at[idx_v.at[blk * K + j]],
                rows_v.at[pl.ds(j * IPS, IPS)],
                gsem,
            )
            for j in range(K)
        ]
        for d in descs:
            d.wait()
        pltpu.sync_copy(rows_v, out_hbm.at[pl.ds(out_base + blk * RPB, RPB)])
        return carry

    lax.fori_loop(0, NBLK, blk_body, 0)


def kernel(x, table):
    x_r = x.reshape(NW, NROWCH, IPS).astype(jnp.int32)
    out = _emb_lookup(x_r, table)
    return out.reshape(x.shape[0], x.shape[1], EMBED)
